# baseline (speedup 1.0000x reference)
import functools

import jax
import jax.numpy as jnp
from jax import lax
from jax.experimental import pallas as pl
from jax.experimental.pallas import tpu as pltpu
from jax.experimental.pallas import tpu_sc as plsc

_PW = 2.0
_ALPHA = 0.75
_HNM = 10
_TVA = 0.3
_TVB = 0.7
_SC_N = 2


def _rsum(x):
    return jnp.sum(jnp.sum(x, axis=0))


def _focal_map(praw, t):
    p = jnp.clip(praw, 1e-6, 1 - 1e-6)
    is_pos = t == 1.0
    p_t = jnp.where(is_pos, p, 1.0 - p)
    q = 1.0 - p_t
    coef = jnp.where(is_pos, _ALPHA * _PW, 1.0 - _ALPHA)
    focal = -(coef * q * q) * jnp.log(p_t)
    return p, is_pos, focal


def _log16(y):
    bits = lax.bitcast_convert_type(y, jnp.int32)
    e = lax.shift_right_arithmetic(bits, 23) - 127
    m = lax.bitcast_convert_type((bits & 0x007FFFFF) | 0x3F800000,
                                 jnp.float32)
    big = m > 1.41421356
    m = jnp.where(big, m * 0.5, m)
    e = e + jnp.where(big, 1, 0)
    z = m - 1.0
    p = 7.0376836292e-2
    for c in (-1.1514610310e-1, 1.1676998740e-1, -1.2420140846e-1,
              1.4249322787e-1, -1.6668057665e-1, 2.0000714765e-1,
              -2.4999993993e-1, 3.3333331174e-1):
        p = p * z + c
    z2 = z * z
    r = z - 0.5 * z2 + z * z2 * p
    return r + e.astype(jnp.float32) * 0.6931471805599453


def _make_sc_huber(sc_n, H, W):
    info = plsc.get_sparse_core_info()
    nc = info.num_cores
    nw = nc * info.num_subcores
    rows_w = sc_n * H // nw
    ch_rows = min(32, rows_w)
    n_ch = rows_w // ch_rows
    w_per_s = H // rows_w

    mesh = plsc.VectorSubcoreMesh(core_axis_name="c", subcore_axis_name="s")

    @functools.partial(
        pl.kernel, mesh=mesh,
        out_type=jax.ShapeDtypeStruct((nw * 32,), jnp.float32),
        scratch_types=[pltpu.VMEM((ch_rows, W), jnp.float32),
                       pltpu.VMEM((ch_rows, W), jnp.float32),
                       pltpu.VMEM((ch_rows, W), jnp.float32),
                       pltpu.VMEM((32,), jnp.float32)],
    )
    def sc_huber(prob_hbm, rlog_hbm, rsp_hbm, out_hbm, bp, bl, br, ob):
        wid = lax.axis_index("s") * nc + lax.axis_index("c")
        smp = wid // w_per_s
        row0 = (wid % w_per_s) * rows_w
        hw = jnp.zeros((16,), jnp.float32)
        ww = jnp.zeros((16,), jnp.float32)

        def inner(j, carry):
            h, w = carry
            r = j // (W // 16)
            o = (j % (W // 16)) * 16
            pv = bp[r, pl.ds(o, 16)]
            rl = bl[r, pl.ds(o, 16)]
            rs = br[r, pl.ds(o, 16)]
            lt = _log16(1.0 + jnp.maximum(rs, 0.0))
            gate = jnp.where((pv > 0.1) | (rs > 1.0), 1.0, 0.0)
            heavy = jnp.where(rs >= 50.0, 3.0, 0.0)
            wt = gate * (1.0 + heavy)
            d = rl - lt
            ad = jnp.abs(d)
            hb = jnp.where(ad < 1.0, 0.5 * d * d, ad - 0.5)
            return h + hb * wt, w + wt

        for ch in range(n_ch):
            r0 = row0 + ch * ch_rows
            pltpu.sync_copy(prob_hbm.at[smp, pl.ds(r0, ch_rows), :], bp)
            pltpu.sync_copy(rlog_hbm.at[smp, pl.ds(r0, ch_rows), :], bl)
            pltpu.sync_copy(rsp_hbm.at[smp, pl.ds(r0, ch_rows), :], br)
            hw, ww = lax.fori_loop(0, ch_rows * W // 16, inner, (hw, ww))

        ob[pl.ds(0, 16)] = hw
        ob[pl.ds(16, 16)] = ww
        pltpu.sync_copy(ob, out_hbm.at[pl.ds(wid * 32, 32)])

    return sc_huber


def _body(prob_ref, label_ref, rlog_ref, rsp_ref, pp_ref, pt_ref, mu_ref,
          std_ref, out_ref):
    b = pl.program_id(0)
    praw = prob_ref[0]
    t = label_ref[0]

    p, _, focal = _focal_map(praw, t)
    n_pos_f = _rsum(t)
    sum_pos = _rsum(focal * t)
    neg_all = _rsum(focal) - sum_pos

    tp = _rsum(p * t)
    fp = _rsum(p) - tp
    fn = n_pos_f - tp
    tv_b = 1.0 - (tp + 1.0) / (tp + _TVA * fp + _TVB * fn + 1.0)

    out_ref[0, 0, 0] = sum_pos
    out_ref[0, 0, 1] = tv_b
    out_ref[0, 0, 5] = n_pos_f
    out_ref[0, 0, 6] = neg_all

    @pl.when(b >= _SC_N)
    def _huber():
        r = rsp_ref[0]
        rlt = jnp.log(1.0 + jnp.maximum(r, 0.0))
        gate = jnp.logical_or(praw > 0.1, r > 1.0).astype(jnp.float32)
        heavy = (r >= 50.0).astype(jnp.float32)
        w = gate * (1.0 + 3.0 * heavy)
        d = rlog_ref[0] - rlt
        ad = jnp.abs(d)
        hub = jnp.where(ad < 1.0, 0.5 * d * d, ad - 0.5)
        out_ref[0, 0, 2] = _rsum(hub * w)
        out_ref[0, 0, 3] = _rsum(w)

    @pl.when(b < _SC_N)
    def _huber0():
        out_ref[0, 0, 2] = 0.0
        out_ref[0, 0, 3] = 0.0

    @pl.when(b == 0)
    def _aux():
        norm = (pt_ref[...] - mu_ref[...]) / (std_ref[...] + 1e-6)
        norm = jnp.where(jnp.isnan(norm), 0.0, norm)
        out_ref[0, 0, 4] = jnp.mean((pp_ref[...] - norm) ** 2)

    @pl.when(b != 0)
    def _aux0():
        out_ref[0, 0, 4] = 0.0


def _fb_body(prob_ref, label_ref, out_ref):
    praw = prob_ref[0]
    t = label_ref[0]
    H, W = praw.shape
    N = H * W
    _, is_pos, focal = _focal_map(praw, t)
    n_pos_i = _rsum(t).astype(jnp.int32)
    n_neg_i = N - n_pos_i
    k = jnp.minimum(n_pos_i * _HNM, n_neg_i)

    vals = jnp.where(is_pos, -1.0, focal)
    vbits = lax.bitcast_convert_type(vals, jnp.int32)

    def step(_, lh):
        lo, hi = lh
        mid = lo + (hi - lo + 1) // 2
        cnt = jnp.sum(jnp.sum((vbits >= mid).astype(jnp.int32), axis=0))
        take = cnt >= k
        return (jnp.where(take, mid, lo), jnp.where(take, hi, mid - 1))

    lo, _ = lax.fori_loop(0, 31, step, (jnp.int32(0), jnp.int32(0x7F7FFFFF)))
    gt = vbits > lo
    eq = vbits == lo
    cnt_gt = _rsum(gt.astype(jnp.float32))
    cnt_eq = jnp.maximum(_rsum(eq.astype(jnp.float32)), 1.0)
    sum_gt = _rsum(jnp.where(gt, focal, 0.0))
    sum_eq = _rsum(jnp.where(eq, focal, 0.0))
    out_ref[0, 0, 0] = (sum_gt
                        + (k.astype(jnp.float32) - cnt_gt) * sum_eq / cnt_eq)


def kernel(prob_map, rain_logit, pred_phys, label_map, rain_max_true,
           rain_spatial_true, phys_targets, phys_mu, phys_std):
    B, H, W = prob_map.shape
    N = H * W
    P = pred_phys.shape[1]
    mu_b = jnp.broadcast_to(phys_mu[None, :], (B, P))
    std_b = jnp.broadcast_to(phys_std[None, :], (B, P))

    sc_out = _make_sc_huber(_SC_N, H, W)(
        prob_map, rain_logit, rain_spatial_true)

    img = pl.BlockSpec((1, H, W), lambda b: (b, 0, 0))
    img_clamp = pl.BlockSpec((1, H, W),
                             lambda b: (jnp.maximum(b, _SC_N), 0, 0))
    small = pl.BlockSpec((B, P), lambda b: (0, 0))
    stats = pl.pallas_call(
        _body,
        grid=(B,),
        in_specs=[img, img, img_clamp, img_clamp, small, small, small, small],
        out_specs=pl.BlockSpec((1, 1, 8), lambda b: (b, 0, 0),
                               memory_space=pltpu.SMEM),
        out_shape=jax.ShapeDtypeStruct((B, 1, 8), jnp.float32),
    )(prob_map, label_map, rain_logit, rain_spatial_true,
      pred_phys, phys_targets, mu_b, std_b)

    stats = stats[:, 0, :]
    sum_pos = stats[:, 0]
    tv_b = stats[:, 1]
    n_pos_f = stats[:, 5]
    neg_all = stats[:, 6]

    n_pos_i = n_pos_f.astype(jnp.int32)
    n_neg_i = N - n_pos_i
    n_hard_i = jnp.minimum(n_pos_i * _HNM, n_neg_i)
    common = n_hard_i == n_neg_i

    def _fallback():
        fb = pl.pallas_call(
            _fb_body,
            grid=(B,),
            in_specs=[img, img],
            out_specs=pl.BlockSpec((1, 1, 8), lambda b: (b, 0, 0),
                                   memory_space=pltpu.SMEM),
            out_shape=jax.ShapeDtypeStruct((B, 1, 8), jnp.float32),
        )(prob_map, label_map)
        return jnp.where(common, neg_all, fb[:, 0, 0])

    sum_hard = lax.cond(jnp.all(common), lambda: neg_all, _fallback)

    fl = jnp.mean((sum_pos + sum_hard)
                  / (n_pos_f + n_hard_i.astype(jnp.float32)))
    tv = jnp.mean(tv_b)
    sco = sc_out.reshape(-1, 2, 16)
    reg_num = jnp.sum(stats[:, 2]) + jnp.sum(sco[:, 0, :])
    reg_den = jnp.sum(stats[:, 3]) + jnp.sum(sco[:, 1, :])
    reg = reg_num / jnp.maximum(reg_den, 1.0)
    aux = stats[0, 4]
    total = fl + 0.5 * tv + 1.0 * reg + 0.1 * aux
    return (total, fl, tv, reg, aux)


# speedup vs baseline: 1.3818x; 1.3818x over previous
import jax
import jax.numpy as jnp
from jax import lax
from jax.experimental import pallas as pl
from jax.experimental.pallas import tpu as pltpu

_PW = 2.0
_ALPHA = 0.75
_HNM = 10
_TVA = 0.3
_TVB = 0.7


def _rsum(x):
    return jnp.sum(jnp.sum(x, axis=0))


def _focal_map(praw, t):
    p = jnp.clip(praw, 1e-6, 1 - 1e-6)
    is_pos = t == 1.0
    p_t = jnp.where(is_pos, p, 1.0 - p)
    q = 1.0 - p_t
    coef = jnp.where(is_pos, _ALPHA * _PW, 1.0 - _ALPHA)
    focal = -(coef * q * q) * jnp.log(p_t)
    return p, is_pos, focal


def _body(prob_ref, label_ref, rlog_ref, rsp_ref, pp_ref, pt_ref, mu_ref,
          std_ref, out_ref):
    b = pl.program_id(0)
    praw = prob_ref[0]
    t = label_ref[0]

    p, _, focal = _focal_map(praw, t)
    n_pos_f = _rsum(t)
    sum_focal = _rsum(focal)

    tp = _rsum(p * t)
    fp = _rsum(p) - tp
    fn = n_pos_f - tp
    tv_b = 1.0 - (tp + 1.0) / (tp + _TVA * fp + _TVB * fn + 1.0)

    r = rsp_ref[0]
    rlt = jnp.log(1.0 + jnp.maximum(r, 0.0))
    gate = jnp.logical_or(praw > 0.1, r > 1.0).astype(jnp.float32)
    heavy = (r >= 50.0).astype(jnp.float32)
    w = gate + 3.0 * heavy
    d = rlog_ref[0] - rlt
    ad = jnp.abs(d)
    hub = jnp.where(ad < 1.0, 0.5 * d * d, ad - 0.5)

    out_ref[0, 0, 0] = sum_focal
    out_ref[0, 0, 1] = tv_b
    out_ref[0, 0, 2] = _rsum(hub * w)
    out_ref[0, 0, 3] = _rsum(w)
    out_ref[0, 0, 5] = n_pos_f

    @pl.when(b == 0)
    def _aux():
        norm = (pt_ref[...] - mu_ref[...]) / (std_ref[...] + 1e-6)
        norm = jnp.where(jnp.isnan(norm), 0.0, norm)
        out_ref[0, 0, 4] = jnp.mean((pp_ref[...] - norm) ** 2)

    @pl.when(b != 0)
    def _aux0():
        out_ref[0, 0, 4] = 0.0


def _fb_body(prob_ref, label_ref, out_ref):
    praw = prob_ref[0]
    t = label_ref[0]
    H, W = praw.shape
    N = H * W
    _, is_pos, focal = _focal_map(praw, t)
    n_pos_i = _rsum(t).astype(jnp.int32)
    n_neg_i = N - n_pos_i
    k = jnp.minimum(n_pos_i * _HNM, n_neg_i)

    vals = jnp.where(is_pos, -1.0, focal)
    vbits = lax.bitcast_convert_type(vals, jnp.int32)

    def step(_, lh):
        lo, hi = lh
        mid = lo + (hi - lo + 1) // 2
        cnt = jnp.sum(jnp.sum((vbits >= mid).astype(jnp.int32), axis=0))
        take = cnt >= k
        return (jnp.where(take, mid, lo), jnp.where(take, hi, mid - 1))

    lo, _ = lax.fori_loop(0, 31, step, (jnp.int32(0), jnp.int32(0x7F7FFFFF)))
    gt = vbits > lo
    eq = vbits == lo
    cnt_gt = _rsum(gt.astype(jnp.float32))
    cnt_eq = jnp.maximum(_rsum(eq.astype(jnp.float32)), 1.0)
    sum_gt = _rsum(jnp.where(gt, focal, 0.0))
    sum_eq = _rsum(jnp.where(eq, focal, 0.0))
    out_ref[0, 0, 0] = (sum_gt
                        + (k.astype(jnp.float32) - cnt_gt) * sum_eq / cnt_eq)
    out_ref[0, 0, 1] = _rsum(focal * t)


def kernel(prob_map, rain_logit, pred_phys, label_map, rain_max_true,
           rain_spatial_true, phys_targets, phys_mu, phys_std):
    B, H, W = prob_map.shape
    N = H * W
    P = pred_phys.shape[1]
    mu_b = jnp.broadcast_to(phys_mu[None, :], (B, P))
    std_b = jnp.broadcast_to(phys_std[None, :], (B, P))

    img = pl.BlockSpec((1, H, W), lambda b: (b, 0, 0))
    small = pl.BlockSpec((B, P), lambda b: (0, 0))
    stats = pl.pallas_call(
        _body,
        grid=(B,),
        in_specs=[img, img, img, img, small, small, small, small],
        out_specs=pl.BlockSpec((1, 1, 8), lambda b: (b, 0, 0),
                               memory_space=pltpu.SMEM),
        out_shape=jax.ShapeDtypeStruct((B, 1, 8), jnp.float32),
        compiler_params=pltpu.CompilerParams(
            dimension_semantics=("parallel",)),
    )(prob_map, label_map, rain_logit, rain_spatial_true,
      pred_phys, phys_targets, mu_b, std_b)

    stats = stats[:, 0, :]
    sum_focal = stats[:, 0]
    tv_b = stats[:, 1]
    n_pos_f = stats[:, 5]

    n_pos_i = n_pos_f.astype(jnp.int32)
    n_neg_i = N - n_pos_i
    n_hard_i = jnp.minimum(n_pos_i * _HNM, n_neg_i)
    common = n_hard_i == n_neg_i

    def _fallback():
        fb = pl.pallas_call(
            _fb_body,
            grid=(B,),
            in_specs=[img, img],
            out_specs=pl.BlockSpec((1, 1, 8), lambda b: (b, 0, 0),
                                   memory_space=pltpu.SMEM),
            out_shape=jax.ShapeDtypeStruct((B, 1, 8), jnp.float32),
        )(prob_map, label_map)
        return jnp.where(common, sum_focal, fb[:, 0, 1] + fb[:, 0, 0])

    numer = lax.cond(jnp.all(common), lambda: sum_focal, _fallback)

    fl = jnp.mean(numer / (n_pos_f + n_hard_i.astype(jnp.float32)))
    tv = jnp.mean(tv_b)
    reg = jnp.sum(stats[:, 2]) / jnp.maximum(jnp.sum(stats[:, 3]), 1.0)
    aux = stats[0, 4]
    total = fl + 0.5 * tv + 1.0 * reg + 0.1 * aux
    return (total, fl, tv, reg, aux)
